# baseline (speedup 1.0000x reference)
import functools

import jax
import jax.numpy as jnp
from jax import lax
from jax.experimental import pallas as pl
from jax.experimental.pallas import tpu as pltpu
from jax.experimental.pallas import tpu_sc as plsc

NC = 2
NS = 16
NW = NC * NS

BB = 128
NBUF = 4


def _make_gather(batch, hist, vocab, dim):
  n_units = (batch // BB) * hist
  assert n_units % (NW * NBUF) == 0
  per_w = n_units // NW
  na = dim // 8
  nt = batch // BB

  mesh = plsc.VectorSubcoreMesh(core_axis_name="c", subcore_axis_name="s")

  @functools.partial(
      pl.kernel,
      out_type=jax.ShapeDtypeStruct((hist, na, nt, 8, BB), jnp.float32),
      mesh=mesh,
      scratch_types=(
          [pltpu.VMEM((BB,), jnp.int32) for _ in range(NBUF)]
          + [pltpu.VMEM((BB, dim), jnp.float32) for _ in range(NBUF)]
          + [pltpu.VMEM((na, 8, BB), jnp.float32) for _ in range(NBUF)]
          + [pltpu.SemaphoreType.DMA for _ in range(3 * NBUF)]
      ),
      compiler_params=pltpu.CompilerParams(
          use_tc_tiling_on_sc=False, needs_layout_passes=False
      ),
  )
  def gather_kernel(tok_hbm, table_hbm, out_hbm, *bufs):
    idx = bufs[:NBUF]
    rows = bufs[NBUF:2 * NBUF]
    tout = bufs[2 * NBUF:3 * NBUF]
    isem = bufs[3 * NBUF:4 * NBUF]
    gsem = bufs[4 * NBUF:5 * NBUF]
    wsem = bufs[5 * NBUF:]
    wid = lax.axis_index("s") * NC + lax.axis_index("c")
    base = wid * per_w
    iota = lax.iota(jnp.int32, 16)

    def fire_idx(u, b):
      h = u // nt
      t = u % nt
      pltpu.async_copy(tok_hbm.at[h, pl.ds(t * BB, BB)], idx[b], isem[b])

    def wait_idx(b):
      pltpu.make_async_copy(tok_hbm.at[0, pl.ds(0, BB)], idx[b], isem[b]).wait()

    def fire_gather(b):
      pltpu.async_copy(table_hbm.at[idx[b]], rows[b], gsem[b])

    def wait_gather(b):
      pltpu.make_async_copy(table_hbm.at[idx[b]], rows[b], gsem[b]).wait()

    def transpose(b):
      @pl.loop(0, dim)
      def _d(d):
        a = d // 8
        s = d % 8
        dcol = iota * 0 + d
        for j0 in range(BB // 16):
          v = plsc.load_gather(rows[b], [j0 * 16 + iota, dcol])
          tout[b][a, s, pl.ds(j0 * 16, 16)] = v

    def fire_write(u, b):
      h = u // nt
      t = u % nt
      pltpu.async_copy(tout[b], out_hbm.at[h, :, t], wsem[b])

    def wait_write(b):
      pltpu.make_async_copy(tout[b], out_hbm.at[0, :, 0], wsem[b]).wait()

    for b in range(NBUF):
      fire_idx(base + b, b)

    @pl.loop(0, per_w, step=NBUF)
    def _units(k):
      for b in range(NBUF):
        u = base + k + b
        wait_idx(b)
        fire_gather(b)
        wait_gather(b)
        transpose(b)
        fire_write(u, b)
        wait_write(b)

        @pl.when(k + b + NBUF < per_w)
        def _():
          fire_idx(u + NBUF, b)

  return gather_kernel


def kernel(token_ids, weight):
  batch, hist = token_ids.shape
  vocab, dim = weight.shape
  tok_t = token_ids.T
  p = _make_gather(batch, hist, vocab, dim)(tok_t, weight)
  o = p.transpose(0, 1, 3, 2, 4).reshape(hist, dim, batch)
  return o.transpose(2, 0, 1)


# speedup vs baseline: 2.3510x; 2.3510x over previous
import functools

import jax
import jax.numpy as jnp
from jax import lax
from jax.experimental import pallas as pl
from jax.experimental.pallas import tpu as pltpu
from jax.experimental.pallas import tpu_sc as plsc

NC = 2
NS = 16
NW = NC * NS

BB = 128
NBUF = 4


def _make_gather(batch, hist, vocab, dim):
  n_units = (batch // BB) * hist
  assert n_units % (NW * NBUF) == 0
  per_w = n_units // NW
  na = dim // 8
  nt = batch // BB

  mesh = plsc.VectorSubcoreMesh(core_axis_name="c", subcore_axis_name="s")

  @functools.partial(
      pl.kernel,
      out_type=jax.ShapeDtypeStruct((hist, na, nt, 8, BB), jnp.float32),
      mesh=mesh,
      scratch_types=(
          [pltpu.VMEM((BB,), jnp.int32) for _ in range(NBUF)]
          + [pltpu.VMEM((BB, dim), jnp.float32) for _ in range(NBUF)]
          + [pltpu.VMEM((dim, BB), jnp.float32) for _ in range(NBUF)]
          + [pltpu.SemaphoreType.DMA for _ in range(3 * NBUF)]
      ),
      compiler_params=pltpu.CompilerParams(
          use_tc_tiling_on_sc=False, needs_layout_passes=False
      ),
  )
  def gather_kernel(tok_hbm, table_hbm, out_hbm, *bufs):
    idx = bufs[:NBUF]
    rows = bufs[NBUF:2 * NBUF]
    tout = bufs[2 * NBUF:3 * NBUF]
    isem = bufs[3 * NBUF:4 * NBUF]
    gsem = bufs[4 * NBUF:5 * NBUF]
    wsem = bufs[5 * NBUF:]
    wid = lax.axis_index("s") * NC + lax.axis_index("c")
    base = wid * per_w
    iota = lax.iota(jnp.int32, 16)
    rbase = [j0 * 16 + iota for j0 in range(BB // 16)]

    def fire_idx(u, b):
      h = u // nt
      t = u % nt
      pltpu.async_copy(tok_hbm.at[h, pl.ds(t * BB, BB)], idx[b], isem[b])

    def wait_idx(b):
      pltpu.make_async_copy(tok_hbm.at[0, pl.ds(0, BB)], idx[b], isem[b]).wait()

    def fire_gather(b):
      pltpu.async_copy(table_hbm.at[idx[b]], rows[b], gsem[b])

    def wait_gather(b):
      pltpu.make_async_copy(table_hbm.at[idx[b]], rows[b], gsem[b]).wait()

    def transpose(b):
      rows_b = rows[b]
      tout_b = tout[b]

      @functools.partial(plsc.parallel_loop, 0, dim, unroll=4)
      def _d(d):
        dcol = iota * 0 + d
        vs = [
            plsc.load_gather(rows_b, [rbase[j0], dcol])
            for j0 in range(BB // 16)
        ]
        for j0 in range(BB // 16):
          tout_b[d, pl.ds(j0 * 16, 16)] = vs[j0]

    def fire_write(u, b):
      h = u // nt
      t = u % nt
      for a in range(na):
        pltpu.async_copy(
            tout[b].at[pl.ds(a * 8, 8)], out_hbm.at[h, a, t], wsem[b]
        )

    def wait_write(b):
      for a in range(na):
        pltpu.make_async_copy(
            tout[b].at[pl.ds(a * 8, 8)], out_hbm.at[0, a, 0], wsem[b]
        ).wait()

    for b in range(NBUF):
      fire_idx(base + b, b)

    @pl.loop(0, per_w, step=NBUF)
    def _units(k):
      for b in range(NBUF):
        u = base + k + b
        wait_idx(b)
        fire_gather(b)
        wait_gather(b)
        transpose(b)
        fire_write(u, b)
        wait_write(b)

        @pl.when(k + b + NBUF < per_w)
        def _():
          fire_idx(u + NBUF, b)

  return gather_kernel


def kernel(token_ids, weight):
  batch, hist = token_ids.shape
  vocab, dim = weight.shape
  tok_t = token_ids.T
  p = _make_gather(batch, hist, vocab, dim)(tok_t, weight)
  o = p.transpose(0, 1, 3, 2, 4).reshape(hist, dim, batch)
  return o.transpose(2, 0, 1)
